# baseline (speedup 1.0000x reference)
import jax
import jax.numpy as jnp
from jax.experimental import pallas as pl
from jax.experimental.pallas import tpu as pltpu

_B = 4
_CB = 256

_CP = pltpu.CompilerParams(
    dimension_semantics=("parallel",),
    vmem_limit_bytes=64 * 1024 * 1024,
)


def _lrn(x, n_real):
    c = x.shape[-1]
    sq = x * x
    s = sq
    if n_real < c:
        for d in (1, 2):
            s = s + pltpu.roll(sq, shift=c - d, axis=1)
            s = s + pltpu.roll(sq, shift=d, axis=1)
    else:
        lane = jax.lax.broadcasted_iota(jnp.int32, x.shape, 1)
        for d in (1, 2):
            s = s + jnp.where(lane + d < c,
                              pltpu.roll(sq, shift=c - d, axis=1), 0.0)
            s = s + jnp.where(lane - d >= 0,
                              pltpu.roll(sq, shift=d, axis=1), 0.0)
    base = 1.0 + (1e-4 / 5.0) * s
    return x * jnp.exp(-0.75 * jnp.log(base))


def _tap(xb, s, valid):
    m = xb.shape[0]
    r = pltpu.roll(xb, shift=(-s) % m, axis=0)
    return jnp.where(valid, r, jnp.zeros((), xb.dtype))


def _pool_sel(n_out, w_in, w_out, dtype):
    ri = jax.lax.broadcasted_iota(jnp.int32, (_B * n_out, _B * w_in), 0)
    ci = jax.lax.broadcasted_iota(jnp.int32, (_B * n_out, _B * w_in), 1)
    q = ri // n_out
    return (ci == w_in * q + 2 * (ri - n_out * q)).astype(dtype)


def _wmax3(rm):
    m = rm.shape[0]
    return jnp.maximum(jnp.maximum(rm, pltpu.roll(rm, m - 1, axis=0)),
                       pltpu.roll(rm, m - 2, axis=0))


def _trunk_kernel(xc_ref, w1_ref, b1_ref, w2_ref, b2_ref, w3_ref, b3_ref,
                  w4_ref, b4_ref, w5_ref, b5_ref, out_ref,
                  a1_ref, p1_ref, a2_ref, p2_ref):
    B = _B

    xc = xc_ref[0].reshape(28 * B * 28, 25)
    h = jnp.dot(xc, w1_ref[...], preferred_element_type=jnp.float32)
    h = jnp.maximum(h + b1_ref[...], 0.0)
    a1_ref[...] = _lrn(h, 96).astype(jnp.bfloat16).reshape(28, B * 28, 128)

    sel1 = _pool_sel(13, 28, 13, jnp.bfloat16)
    for i in range(13):
        rm = jnp.maximum(jnp.maximum(a1_ref[2 * i], a1_ref[2 * i + 1]),
                         a1_ref[2 * i + 2])
        p1_ref[i] = jnp.dot(sel1, _wmax3(rm),
                            preferred_element_type=jnp.float32
                            ).astype(jnp.bfloat16)

    m2 = 13 * B * 13
    p1 = p1_ref[...].reshape(m2, 128)
    r2 = jax.lax.broadcasted_iota(jnp.int32, (m2, 1), 0)
    i2 = r2 // (B * 13)
    j2 = r2 % 13
    acc = None
    for kh in range(5):
        for kw in range(5):
            valid = ((i2 >= 2 - kh) & (i2 < 15 - kh) &
                     (j2 >= 2 - kw) & (j2 < 15 - kw))
            t = _tap(p1, (kh - 2) * (B * 13) + (kw - 2), valid)
            d = jnp.dot(t, w2_ref[kh * 5 + kw],
                        preferred_element_type=jnp.float32)
            acc = d if acc is None else acc + d
    y2 = jnp.maximum(acc + b2_ref[...], 0.0)
    a2_ref[...] = _lrn(y2, 256).astype(jnp.bfloat16).reshape(13, B * 13, 256)

    sel2 = _pool_sel(6, 13, 6, jnp.bfloat16)
    for i in range(6):
        rm = jnp.maximum(jnp.maximum(a2_ref[2 * i], a2_ref[2 * i + 1]),
                         a2_ref[2 * i + 2])
        p2_ref[i] = jnp.dot(sel2, _wmax3(rm),
                            preferred_element_type=jnp.float32
                            ).astype(jnp.bfloat16)

    m3 = 6 * B * 6
    r3 = jax.lax.broadcasted_iota(jnp.int32, (m3, 1), 0)
    i3 = r3 // (B * 6)
    j3 = r3 % 6

    def conv3x3(src, w_ref, b_ref):
        acc = None
        for kh in range(3):
            for kw in range(3):
                valid = ((i3 >= 1 - kh) & (i3 < 7 - kh) &
                         (j3 >= 1 - kw) & (j3 < 7 - kw))
                t = _tap(src, (kh - 1) * (B * 6) + (kw - 1), valid)
                d = jnp.dot(t, w_ref[kh * 3 + kw],
                            preferred_element_type=jnp.float32)
                acc = d if acc is None else acc + d
        return jnp.maximum(acc + b_ref[...], 0.0)

    y3 = conv3x3(p2_ref[...].reshape(m3, 256), w3_ref, b3_ref).astype(jnp.bfloat16)
    y4 = conv3x3(y3, w4_ref, b4_ref).astype(jnp.bfloat16)
    a5 = conv3x3(y4, w5_ref, b5_ref).astype(jnp.bfloat16).reshape(6, B * 6, 256)

    sel3 = _pool_sel(2, 6, 2, jnp.bfloat16)
    for i in range(2):
        rm = jnp.maximum(jnp.maximum(a5[2 * i], a5[2 * i + 1]), a5[2 * i + 2])
        out_ref[0, i] = jnp.dot(sel3, _wmax3(rm),
                                preferred_element_type=jnp.float32
                                ).astype(jnp.bfloat16)


def _trunk(xcol, w1, b1, w2, b2, w3, b3, w4, b4, w5, b5):
    nb = xcol.shape[0]
    B = _B
    return pl.pallas_call(
        _trunk_kernel,
        out_shape=jax.ShapeDtypeStruct((nb, 2, B * 2, 256), jnp.bfloat16),
        grid=(nb,),
        in_specs=[
            pl.BlockSpec((1, 28, B * 28, 25), lambda i: (i, 0, 0, 0)),
            pl.BlockSpec((25, 128), lambda i: (0, 0)),
            pl.BlockSpec((1, 128), lambda i: (0, 0)),
            pl.BlockSpec((25, 128, 256), lambda i: (0, 0, 0)),
            pl.BlockSpec((1, 256), lambda i: (0, 0)),
            pl.BlockSpec((9, 256, 384), lambda i: (0, 0, 0)),
            pl.BlockSpec((1, 384), lambda i: (0, 0)),
            pl.BlockSpec((9, 384, 384), lambda i: (0, 0, 0)),
            pl.BlockSpec((1, 384), lambda i: (0, 0)),
            pl.BlockSpec((9, 384, 256), lambda i: (0, 0, 0)),
            pl.BlockSpec((1, 256), lambda i: (0, 0)),
        ],
        out_specs=pl.BlockSpec((1, 2, B * 2, 256), lambda i: (i, 0, 0, 0)),
        scratch_shapes=[
            pltpu.VMEM((28, B * 28, 128), jnp.bfloat16),
            pltpu.VMEM((13, B * 13, 128), jnp.bfloat16),
            pltpu.VMEM((13, B * 13, 256), jnp.bfloat16),
            pltpu.VMEM((6, B * 6, 256), jnp.bfloat16),
        ],
        compiler_params=_CP,
    )(xcol, w1, b1, w2, b2, w3, b3, w4, b4, w5, b5)


def _cls_kernel(x_ref, w1_ref, b1_ref, w2_ref, b2_ref, out_ref):
    h = jnp.dot(x_ref[...], w1_ref[...], preferred_element_type=jnp.float32)
    h = jnp.maximum(h + b1_ref[...], 0.0)
    y = jnp.dot(h.astype(jnp.bfloat16), w2_ref[...],
                preferred_element_type=jnp.float32) + b2_ref[...]
    m = jnp.max(y, axis=-1, keepdims=True)
    e = jnp.exp(y - m)
    out_ref[...] = e / jnp.sum(e, axis=-1, keepdims=True)


def _cls(x, w1, b1, w2, b2):
    n = x.shape[0]
    return pl.pallas_call(
        _cls_kernel,
        out_shape=jax.ShapeDtypeStruct((n, 10), jnp.float32),
        grid=(n // _CB,),
        in_specs=[
            pl.BlockSpec((_CB, 1024), lambda i: (i, 0)),
            pl.BlockSpec((1024, 2304), lambda i: (0, 0)),
            pl.BlockSpec((1, 2304), lambda i: (0, 0)),
            pl.BlockSpec((2304, 10), lambda i: (0, 0)),
            pl.BlockSpec((1, 10), lambda i: (0, 0)),
        ],
        out_specs=pl.BlockSpec((_CB, 10), lambda i: (i, 0)),
        compiler_params=_CP,
    )(x, w1, b1, w2, b2)


@jax.jit
def _forward(x, conv1_w, conv1_b, conv2_w, conv2_b, conv3_w, conv3_b,
             conv4_w, conv4_b, conv5_w, conv5_b, lin1_w, lin1_b,
             lin2_w, lin2_b):
    n = x.shape[0]
    B = _B
    npad = -(-n // B) * B
    x3 = x.astype(jnp.float32).reshape(n, 28, 28)
    if npad != n:
        x3 = jnp.pad(x3, ((0, npad - n), (0, 0), (0, 0)))
    xp = jnp.pad(x3, ((0, 0), (2, 2), (2, 2)))
    xcol = jnp.stack([xp[:, kh:kh + 28, kw:kw + 28]
                      for kh in range(5) for kw in range(5)], axis=-1)
    xcol = (xcol.reshape(npad // B, B, 28, 28, 25)
            .transpose(0, 2, 1, 3, 4)
            .reshape(npad // B, 28, B * 28, 25).astype(jnp.bfloat16))

    w1 = jnp.pad(conv1_w.reshape(25, 96), ((0, 0), (0, 32))).astype(jnp.bfloat16)
    b1 = jnp.pad(conv1_b, (0, 32)).reshape(1, 128).astype(jnp.float32)
    w2 = (jnp.pad(conv2_w, ((0, 0), (0, 0), (0, 32), (0, 0)))
          .reshape(25, 128, 256).astype(jnp.bfloat16))
    b2 = conv2_b.reshape(1, 256).astype(jnp.float32)
    w3 = conv3_w.reshape(9, 256, 384).astype(jnp.bfloat16)
    b3 = conv3_b.reshape(1, 384).astype(jnp.float32)
    w4 = conv4_w.reshape(9, 384, 384).astype(jnp.bfloat16)
    b4 = conv4_b.reshape(1, 384).astype(jnp.float32)
    w5 = conv5_w.reshape(9, 384, 256).astype(jnp.bfloat16)
    b5 = conv5_b.reshape(1, 256).astype(jnp.float32)

    f = _trunk(xcol, w1, b1, w2, b2, w3, b3, w4, b4, w5, b5)
    f = (f.reshape(npad // B, 2, B, 2, 256).transpose(0, 2, 1, 3, 4)
         .reshape(npad, 1024))
    n2 = -(-npad // _CB) * _CB
    if n2 != npad:
        f = jnp.pad(f, ((0, n2 - npad), (0, 0)))

    wl1 = (lin1_w.reshape(256, 4, 2304).transpose(1, 0, 2)
           .reshape(1024, 2304).astype(jnp.bfloat16))
    bl1 = lin1_b.reshape(1, 2304).astype(jnp.float32)
    wl2 = lin2_w.astype(jnp.bfloat16)
    bl2 = lin2_b.reshape(1, 10).astype(jnp.float32)
    probs = _cls(f, wl1, bl1, wl2, bl2)
    return probs[:n]


def kernel(x, conv1_w, conv1_b, conv2_w, conv2_b, conv3_w, conv3_b,
           conv4_w, conv4_b, conv5_w, conv5_b, lin1_w, lin1_b,
           lin2_w, lin2_b):
    return _forward(x, conv1_w, conv1_b, conv2_w, conv2_b, conv3_w, conv3_b,
                    conv4_w, conv4_b, conv5_w, conv5_b, lin1_w, lin1_b,
                    lin2_w, lin2_b)
